# baseline (speedup 1.0000x reference)
import functools

import jax
import jax.numpy as jnp
from jax import lax
from jax.experimental import pallas as pl
from jax.experimental.pallas import tpu as pltpu
from jax.experimental.pallas import tpu_sc as plsc

_B = 16384
_F = 26
_D = 16
_FIELD = 38462
_NROWS = _F * _FIELD
_TAIL0 = (_NROWS // 128) * 128
_NTAIL = _NROWS - _TAIL0
_W = 38656
_NV = _B // 16


def _win(f):
    lo = f * _FIELD
    c0 = (lo // 128) * 128
    if c0 + _W <= _NROWS:
        return c0, _W, lo - c0
    return c0, _TAIL0 - c0, lo - c0


def _emit_half(fields, t2, xs, out3, tbl_v, idx0, out_v, tail_v,
               spt0, spo0, sem_tbl, sem_idx, sem_pull, ss0, s):
    nf = len(fields)
    is0 = s == 0

    def tbl_cp(k):
        c0, w, _ = _win(fields[k])
        return pltpu.make_async_copy(
            t2.at[:, pl.ds(c0, w)], spt0.at[:, pl.ds(0, w)], sem_tbl
        )

    def idx_cp(k):
        return pltpu.make_async_copy(xs.at[fields[k], :], idx0, sem_idx)

    def store_cp(k):
        return pltpu.make_async_copy(spo0, out3.at[fields[k]], ss0)

    def pull_cp():
        return pltpu.make_async_copy(spt0.at[s], tbl_v, sem_pull)

    @pl.when(is0)
    def _():
        tbl_cp(0).start()

    idx_cp(0).start()

    for k in range(nf):
        f = fields[k]
        _, _, shift = _win(f)

        @pl.when(is0)
        def _():
            tbl_cp(k).wait()

        plsc.subcore_barrier()

        pull_cp().start()

        if k >= 1:
            if k >= 2:
                @pl.when(is0)
                def _():
                    store_cp(k - 2).wait()

                plsc.subcore_barrier()

            pltpu.sync_copy(out_v, spo0.at[s])
            plsc.subcore_barrier()

            @pl.when(is0)
            def _():
                store_cp(k - 1).start()

        idx_cp(k).wait()
        pull_cp().wait()
        plsc.subcore_barrier()

        if k + 1 < nf:
            @pl.when(is0)
            def _():
                tbl_cp(k + 1).start()

        iv = idx0
        if f == _F - 1:
            lim = _TAIL0 - f * _FIELD
            dsplat = jnp.zeros((16,), jnp.int32) + s

            def gstep(i, cc):
                for u in range(16):
                    sl = pl.ds((i * 16 + u) * 16, 16)
                    xv = iv[sl]
                    va = plsc.load_gather(tbl_v, [xv + shift])
                    ti = lax.max(xv - lim, 0)
                    vb = plsc.load_gather(tail_v, [dsplat, ti])
                    out_v[sl] = jnp.where(xv >= lim, vb, va)
                return cc

            lax.fori_loop(0, _NV // 16, gstep, 0)
        else:
            def gstep(i, cc):
                for u in range(16):
                    sl = pl.ds((i * 16 + u) * 16, 16)
                    out_v[sl] = plsc.load_gather(tbl_v, [iv[sl] + shift])
                return cc

            lax.fori_loop(0, _NV // 16, gstep, 0)

        if k + 1 < nf:
            idx_cp(k + 1).start()

    @pl.when(is0)
    def _():
        store_cp(nf - 2).wait()

    plsc.subcore_barrier()
    pltpu.sync_copy(out_v, spo0.at[s])
    plsc.subcore_barrier()

    @pl.when(is0)
    def _():
        store_cp(nf - 1).start()
        store_cp(nf - 1).wait()


def _body(t2, xs, tail, out3, tbl_v, idx0, out_v, tail_v,
          spt0, spo0, sem_tbl, sem_idx, sem_pull, ss0):
    c = lax.axis_index("c")
    s = lax.axis_index("s")

    pltpu.sync_copy(tail, tail_v)

    args = (t2, xs, out3, tbl_v, idx0, out_v, tail_v,
            spt0, spo0, sem_tbl, sem_idx, sem_pull, ss0, s)

    @pl.when(c == 0)
    def _():
        _emit_half(list(range(13)), *args)

    @pl.when(c == 1)
    def _():
        _emit_half(list(range(13, 26)), *args)


@jax.jit
def _embed(x, table):
    t2 = table.T
    xs = x.T
    tail = jnp.pad(t2[:, _TAIL0:], ((0, 0), (0, 128 - _NTAIL)))
    mesh = plsc.VectorSubcoreMesh(core_axis_name="c", subcore_axis_name="s")
    run = functools.partial(
        pl.kernel,
        mesh=mesh,
        out_type=jax.ShapeDtypeStruct((_F, _D, _B), jnp.float32),
        scratch_types=[
            pltpu.VMEM((_W,), jnp.float32),
            pltpu.VMEM((_B,), jnp.int32),
            pltpu.VMEM((_B,), jnp.float32),
            pltpu.VMEM((_D, 128), jnp.float32),
            pltpu.VMEM_SHARED((_D, _W), jnp.float32),
            pltpu.VMEM_SHARED((_D, _B), jnp.float32),
            pltpu.SemaphoreType.DMA,
            pltpu.SemaphoreType.DMA,
            pltpu.SemaphoreType.DMA,
            pltpu.SemaphoreType.DMA,
        ],
        compiler_params=pltpu.CompilerParams(needs_layout_passes=False),
    )(_body)
    out3 = run(t2, xs, tail)
    return out3.transpose(2, 0, 1)


def kernel(x, table):
    return _embed(x, table)


# speedup vs baseline: 1.1685x; 1.1685x over previous
import functools

import jax
import jax.numpy as jnp
from jax import lax
from jax.experimental import pallas as pl
from jax.experimental.pallas import tpu as pltpu
from jax.experimental.pallas import tpu_sc as plsc

_B = 16384
_F = 26
_D = 16
_FIELD = 38462
_NROWS = _F * _FIELD
_TAIL0 = (_NROWS // 128) * 128
_NTAIL = _NROWS - _TAIL0
_W = 38656
_NV = _B // 16


def _win(f):
    lo = f * _FIELD
    c0 = (lo // 128) * 128
    if c0 + _W <= _NROWS:
        return c0, _W, lo - c0
    return c0, _TAIL0 - c0, lo - c0


def _emit_half(fields, t2, xs, out3, tbl_v, ia, ib, out_v, tail_v,
               spt0, spo0, sem_tbl, sem_ia, sem_ib, sem_pull, ss0, s):
    nf = len(fields)
    is0 = s == 0
    _H = _B // 2

    def tbl_cp(k):
        c0, w, _ = _win(fields[k])
        return pltpu.make_async_copy(
            t2.at[:, pl.ds(c0, w)], spt0.at[:, pl.ds(0, w)], sem_tbl
        )

    def idx_cp(k, h):
        buf, sem = (ia, sem_ia) if h == 0 else (ib, sem_ib)
        return pltpu.make_async_copy(
            xs.at[fields[k], pl.ds(h * _H, _H)], buf, sem)

    def store_cp(k):
        return pltpu.make_async_copy(spo0, out3.at[fields[k]], ss0)

    def pull_cp():
        return pltpu.make_async_copy(spt0.at[s], tbl_v, sem_pull)

    @pl.when(is0)
    def _():
        tbl_cp(0).start()

    idx_cp(0, 0).start()
    idx_cp(0, 1).start()

    for k in range(nf):
        f = fields[k]
        _, _, shift = _win(f)

        @pl.when(is0)
        def _():
            tbl_cp(k).wait()

        plsc.subcore_barrier()

        pull_cp().start()

        if k >= 1:
            if k >= 2:
                @pl.when(is0)
                def _():
                    store_cp(k - 2).wait()

                plsc.subcore_barrier()

            pltpu.sync_copy(out_v, spo0.at[s])
            plsc.subcore_barrier()

            @pl.when(is0)
            def _():
                store_cp(k - 1).start()

        pull_cp().wait()
        plsc.subcore_barrier()

        if k + 1 < nf:
            @pl.when(is0)
            def _():
                tbl_cp(k + 1).start()

        if f == _F - 1:
            lim = _TAIL0 - f * _FIELD
            dsplat = jnp.zeros((16,), jnp.int32) + s

            def ghalf(iv, base):
                def gstep(i, cc):
                    for u in range(8):
                        o = (i * 8 + u) * 16
                        xv = iv[pl.ds(o, 16)]
                        va = plsc.load_gather(tbl_v, [xv + shift])
                        ti = lax.max(xv - lim, 0)
                        vb = plsc.load_gather(tail_v, [dsplat, ti])
                        out_v[pl.ds(base + o, 16)] = jnp.where(
                            xv >= lim, vb, va)
                    return cc

                lax.fori_loop(0, _NV // 16, gstep, 0)
        else:
            def ghalf(iv, base):
                def gstep(i, cc):
                    for u in range(8):
                        o = (i * 8 + u) * 16
                        out_v[pl.ds(base + o, 16)] = plsc.load_gather(
                            tbl_v, [iv[pl.ds(o, 16)] + shift])
                    return cc

                lax.fori_loop(0, _NV // 16, gstep, 0)

        idx_cp(k, 0).wait()
        ghalf(ia, 0)
        if k + 1 < nf:
            idx_cp(k + 1, 0).start()
        idx_cp(k, 1).wait()
        ghalf(ib, _H)
        if k + 1 < nf:
            idx_cp(k + 1, 1).start()

    @pl.when(is0)
    def _():
        store_cp(nf - 2).wait()

    plsc.subcore_barrier()
    pltpu.sync_copy(out_v, spo0.at[s])
    plsc.subcore_barrier()

    @pl.when(is0)
    def _():
        store_cp(nf - 1).start()
        store_cp(nf - 1).wait()


def _body(t2, xs, tail, out3, tbl_v, ia, ib, out_v, tail_v,
          spt0, spo0, sem_tbl, sem_ia, sem_ib, sem_pull, ss0):
    c = lax.axis_index("c")
    s = lax.axis_index("s")

    pltpu.sync_copy(tail, tail_v)

    args = (t2, xs, out3, tbl_v, ia, ib, out_v, tail_v,
            spt0, spo0, sem_tbl, sem_ia, sem_ib, sem_pull, ss0, s)

    @pl.when(c == 0)
    def _():
        _emit_half(list(range(13)), *args)

    @pl.when(c == 1)
    def _():
        _emit_half(list(range(13, 26)), *args)


@jax.jit
def _embed(x, table):
    t2 = table.T
    xs = x.T
    tail = jnp.pad(t2[:, _TAIL0:], ((0, 0), (0, 128 - _NTAIL)))
    mesh = plsc.VectorSubcoreMesh(core_axis_name="c", subcore_axis_name="s")
    run = functools.partial(
        pl.kernel,
        mesh=mesh,
        out_type=jax.ShapeDtypeStruct((_F, _D, _B), jnp.float32),
        scratch_types=[
            pltpu.VMEM((_W,), jnp.float32),
            pltpu.VMEM((_B // 2,), jnp.int32),
            pltpu.VMEM((_B // 2,), jnp.int32),
            pltpu.VMEM((_B,), jnp.float32),
            pltpu.VMEM((_D, 128), jnp.float32),
            pltpu.VMEM_SHARED((_D, _W), jnp.float32),
            pltpu.VMEM_SHARED((_D, _B), jnp.float32),
            pltpu.SemaphoreType.DMA,
            pltpu.SemaphoreType.DMA,
            pltpu.SemaphoreType.DMA,
            pltpu.SemaphoreType.DMA,
            pltpu.SemaphoreType.DMA,
        ],
        compiler_params=pltpu.CompilerParams(needs_layout_passes=False),
    )(_body)
    out3 = run(t2, xs, tail)
    return out3.transpose(2, 0, 1)


def kernel(x, table):
    return _embed(x, table)


# speedup vs baseline: 1.1686x; 1.0000x over previous
import functools

import jax
import jax.numpy as jnp
from jax import lax
from jax.experimental import pallas as pl
from jax.experimental.pallas import tpu as pltpu
from jax.experimental.pallas import tpu_sc as plsc

_B = 16384
_F = 26
_D = 16
_FIELD = 38462
_NROWS = _F * _FIELD
_TAIL0 = (_NROWS // 128) * 128
_NTAIL = _NROWS - _TAIL0
_W = 38656
_NV = _B // 16


def _win(f):
    lo = f * _FIELD
    c0 = (lo // 128) * 128
    if c0 + _W <= _NROWS:
        return c0, _W, lo - c0
    return c0, _TAIL0 - c0, lo - c0


def _emit_half(fields, t2, xs, out3, tbl_v, ia, ib, out_v, tail_v,
               spt0, spo0, sem_tbl, sem_ia, sem_ib, sem_pull, ss0, s):
    nf = len(fields)
    is0 = s == 0
    _H = _B // 2

    def tbl_cp(k):
        c0, w, _ = _win(fields[k])
        return pltpu.make_async_copy(
            t2.at[:, pl.ds(c0, w)], spt0.at[:, pl.ds(0, w)], sem_tbl
        )

    def idx_cp(k, h):
        buf, sem = (ia, sem_ia) if h == 0 else (ib, sem_ib)
        return pltpu.make_async_copy(
            xs.at[fields[k], pl.ds(h * _H, _H)], buf, sem)

    def store_cp(k):
        return pltpu.make_async_copy(spo0, out3.at[fields[k]], ss0)

    def pull_cp():
        return pltpu.make_async_copy(spt0.at[s], tbl_v, sem_pull)

    @pl.when(is0)
    def _():
        tbl_cp(0).start()

    idx_cp(0, 0).start()
    idx_cp(0, 1).start()

    for k in range(nf):
        f = fields[k]
        _, _, shift = _win(f)

        @pl.when(is0)
        def _():
            tbl_cp(k).wait()

        plsc.subcore_barrier()

        pull_cp().start()

        if k >= 1:
            if k >= 2:
                @pl.when(is0)
                def _():
                    store_cp(k - 2).wait()

                plsc.subcore_barrier()

            pltpu.sync_copy(out_v, spo0.at[s])
            plsc.subcore_barrier()

            @pl.when(is0)
            def _():
                store_cp(k - 1).start()

        pull_cp().wait()
        plsc.subcore_barrier()

        if k + 1 < nf:
            @pl.when(is0)
            def _():
                tbl_cp(k + 1).start()

        if f == _F - 1:
            lim = _TAIL0 - f * _FIELD
            dsplat = jnp.zeros((16,), jnp.int32) + s

            def ghalf(iv, base):
                def gstep(i, cc):
                    for u in range(8):
                        o = (i * 8 + u) * 16
                        xv = iv[pl.ds(o, 16)]
                        va = plsc.load_gather(tbl_v, [xv + shift])
                        ti = lax.max(xv - lim, 0)
                        vb = plsc.load_gather(tail_v, [dsplat, ti])
                        out_v[pl.ds(base + o, 16)] = jnp.where(
                            xv >= lim, vb, va)
                    return cc

                lax.fori_loop(0, _NV // 16, gstep, 0)
        else:
            def ghalf(iv, base):
                def gstep(i, cc):
                    xvs = [iv[pl.ds((i * 8 + u) * 16, 16)] + shift
                           for u in range(8)]
                    for u in range(8):
                        o = (i * 8 + u) * 16
                        out_v[pl.ds(base + o, 16)] = plsc.load_gather(
                            tbl_v, [xvs[u]])
                    return cc

                lax.fori_loop(0, _NV // 16, gstep, 0)

        idx_cp(k, 0).wait()
        ghalf(ia, 0)
        if k + 1 < nf:
            idx_cp(k + 1, 0).start()
        idx_cp(k, 1).wait()
        ghalf(ib, _H)
        if k + 1 < nf:
            idx_cp(k + 1, 1).start()

    @pl.when(is0)
    def _():
        store_cp(nf - 2).wait()

    plsc.subcore_barrier()
    pltpu.sync_copy(out_v, spo0.at[s])
    plsc.subcore_barrier()

    @pl.when(is0)
    def _():
        store_cp(nf - 1).start()
        store_cp(nf - 1).wait()


def _body(t2, xs, tail, out3, tbl_v, ia, ib, out_v, tail_v,
          spt0, spo0, sem_tbl, sem_ia, sem_ib, sem_pull, ss0):
    c = lax.axis_index("c")
    s = lax.axis_index("s")

    pltpu.sync_copy(tail, tail_v)

    args = (t2, xs, out3, tbl_v, ia, ib, out_v, tail_v,
            spt0, spo0, sem_tbl, sem_ia, sem_ib, sem_pull, ss0, s)

    @pl.when(c == 0)
    def _():
        _emit_half(list(range(13)), *args)

    @pl.when(c == 1)
    def _():
        _emit_half(list(range(13, 26)), *args)


@jax.jit
def _embed(x, table):
    t2 = table.T
    xs = x.T
    tail = jnp.pad(t2[:, _TAIL0:], ((0, 0), (0, 128 - _NTAIL)))
    mesh = plsc.VectorSubcoreMesh(core_axis_name="c", subcore_axis_name="s")
    run = functools.partial(
        pl.kernel,
        mesh=mesh,
        out_type=jax.ShapeDtypeStruct((_F, _D, _B), jnp.float32),
        scratch_types=[
            pltpu.VMEM((_W,), jnp.float32),
            pltpu.VMEM((_B // 2,), jnp.int32),
            pltpu.VMEM((_B // 2,), jnp.int32),
            pltpu.VMEM((_B,), jnp.float32),
            pltpu.VMEM((_D, 128), jnp.float32),
            pltpu.VMEM_SHARED((_D, _W), jnp.float32),
            pltpu.VMEM_SHARED((_D, _B), jnp.float32),
            pltpu.SemaphoreType.DMA,
            pltpu.SemaphoreType.DMA,
            pltpu.SemaphoreType.DMA,
            pltpu.SemaphoreType.DMA,
            pltpu.SemaphoreType.DMA,
        ],
        compiler_params=pltpu.CompilerParams(needs_layout_passes=False),
    )(_body)
    out3 = run(t2, xs, tail)
    return out3.transpose(2, 0, 1)


def kernel(x, table):
    return _embed(x, table)
